# baseline (speedup 1.0000x reference)
import functools

import jax
import jax.numpy as jnp
from jax import lax
from jax.experimental import pallas as pl
from jax.experimental.pallas import tpu as pltpu
from jax.experimental.pallas import tpu_sc as plsc

N_NODES = 10000
D = 128
NC = 2
NS = 16
NW = NC * NS
CHUNK = 112
N_PAD = 10240
RPT = N_PAD // NS
CPT = 92
E_PAD = NW * CPT * CHUNK

_mesh = plsc.VectorSubcoreMesh(core_axis_name="c", subcore_axis_name="s")


@functools.partial(
    pl.kernel,
    mesh=_mesh,
    out_type=jax.ShapeDtypeStruct((NC * N_PAD,), jnp.float32),
    scratch_types=[
        pltpu.VMEM((CPT, CHUNK), jnp.int32),
        pltpu.VMEM((CHUNK,), jnp.float32),
        pltpu.VMEM((640,), jnp.float32),
        pltpu.VMEM_SHARED((N_PAD,), jnp.float32),
        pltpu.SemaphoreType.DMA,
        pltpu.SemaphoreType.DMA,
    ],
)
def _deg_call(dst_hbm, out_hbm, didx_all, ones_v, zero_v, deg_sh, ss0, ss1):
    c = lax.axis_index("c")
    s = lax.axis_index("s")
    wid = s * NC + c
    ss = (ss0, ss1)

    def _fill_zero(i, _):
        zero_v[pl.ds(i * 16, 16)] = jnp.zeros((16,), jnp.float32)
        return 0

    lax.fori_loop(0, 640 // 16, _fill_zero, 0)

    def _fill_one(i, _):
        ones_v[pl.ds(i * 16, 16)] = jnp.ones((16,), jnp.float32)
        return 0

    lax.fori_loop(0, CHUNK // 16, _fill_one, 0)

    pltpu.sync_copy(zero_v.at[pl.ds(0, RPT)], deg_sh.at[pl.ds(s * RPT, RPT)])
    pltpu.sync_copy(dst_hbm.at[wid], didx_all)
    plsc.subcore_barrier()

    def _scat(j, q):
        return pltpu.make_async_copy(
            ones_v, deg_sh.at[didx_all.at[j]], ss[q])

    _scat(0, 0).start(add=True)
    _scat(1, 1).start(add=True)

    def _body(i, _):
        j0 = 2 * i
        _scat(j0 - 2, 0).wait()
        _scat(j0, 0).start(add=True)
        _scat(j0 - 1, 1).wait()
        _scat(j0 + 1, 1).start(add=True)
        return 0

    lax.fori_loop(1, CPT // 2, _body, 0)
    _scat(CPT - 2, 0).wait()
    _scat(CPT - 1, 1).wait()

    plsc.subcore_barrier()
    pltpu.sync_copy(deg_sh.at[pl.ds(s * RPT, RPT)],
                    out_hbm.at[pl.ds(c * N_PAD + s * RPT, RPT)])


@functools.partial(
    pl.kernel,
    mesh=_mesh,
    out_type=jax.ShapeDtypeStruct((NC, N_PAD, D), jnp.float32),
    scratch_types=[
        pltpu.VMEM((CHUNK,), jnp.int32),
        pltpu.VMEM((CHUNK,), jnp.int32),
        pltpu.VMEM((CHUNK,), jnp.int32),
        pltpu.VMEM((CHUNK,), jnp.int32),
        pltpu.VMEM((CHUNK,), jnp.int32),
        pltpu.VMEM((CHUNK,), jnp.int32),
        pltpu.VMEM((CHUNK, D), jnp.float32),
        pltpu.VMEM((CHUNK, D), jnp.float32),
        pltpu.VMEM((CHUNK, D), jnp.float32),
        pltpu.VMEM_SHARED((N_PAD, D), jnp.float32),
        pltpu.SemaphoreType.DMA,
        pltpu.SemaphoreType.DMA,
        pltpu.SemaphoreType.DMA,
        pltpu.SemaphoreType.DMA,
        pltpu.SemaphoreType.DMA,
        pltpu.SemaphoreType.DMA,
        pltpu.SemaphoreType.DMA,
        pltpu.SemaphoreType.DMA,
        pltpu.SemaphoreType.DMA,
        pltpu.SemaphoreType.DMA,
        pltpu.SemaphoreType.DMA,
        pltpu.SemaphoreType.DMA,
        pltpu.SemaphoreType.DMA,
        pltpu.SemaphoreType.DMA,
        pltpu.SemaphoreType.DMA,
    ],
)
def _agg_call(src_hbm, dst_hbm, h_hbm, out_hbm, sid0, sid1, sid2, did0, did1, did2,
              rows0, rows1, rows2, agg_sh,
              si0, si1, si2, sg0, sg1, sg2, sh0, sh1, sh2,
              ss0, ss1, ss2, st0, st1, st2):
    c = lax.axis_index("c")
    s = lax.axis_index("s")
    wid = s * NC + c
    sid = (sid0, sid1, sid2)
    did = (did0, did1, did2)
    rows = (rows0, rows1, rows2)
    si = (si0, si1, si2)
    sg = (sg0, sg1, sg2)
    sh = (sh0, sh1, sh2)
    ss = (ss0, ss1, ss2)
    st = (st0, st1, st2)

    def _zrow(i, _):
        def _zcol(k, _):
            rows0[i, pl.ds(k * 16, 16)] = jnp.zeros((16,), jnp.float32)
            return 0
        return lax.fori_loop(0, D // 16, _zcol, 0)

    lax.fori_loop(0, CHUNK, _zrow, 0)
    for t in range(RPT // CHUNK):
        pltpu.sync_copy(rows0, agg_sh.at[pl.ds(s * RPT + t * CHUNK, CHUNK)])
    if RPT % CHUNK:
        _tail = RPT % CHUNK
        pltpu.sync_copy(
            rows0.at[pl.ds(0, _tail)],
            agg_sh.at[pl.ds(s * RPT + (RPT // CHUNK) * CHUNK, _tail)])
    plsc.subcore_barrier()

    def _idx_start(j, q):
        base = (wid * CPT + j) * CHUNK
        pltpu.async_copy(src_hbm.at[pl.ds(base, CHUNK)], sid[q], si[q])
        pltpu.async_copy(dst_hbm.at[pl.ds(base, CHUNK)], did[q], si[q])

    def _idx_wait(j, q):
        base = (wid * CPT + j) * CHUNK
        pltpu.make_async_copy(src_hbm.at[pl.ds(base, CHUNK)], sid[q], si[q]).wait()
        pltpu.make_async_copy(dst_hbm.at[pl.ds(base, CHUNK)], did[q], si[q]).wait()

    HALF = CHUNK // 2

    def _gat_a(q):
        return pltpu.make_async_copy(
            h_hbm.at[sid[q].at[pl.ds(0, HALF)]],
            rows[q].at[pl.ds(0, HALF)], sg[q])

    def _gat_b(q):
        return pltpu.make_async_copy(
            h_hbm.at[sid[q].at[pl.ds(HALF, HALF)]],
            rows[q].at[pl.ds(HALF, HALF)], sh[q])

    def _gat_start(q):
        _gat_a(q).start()
        _gat_b(q).start()

    def _gat_wait(q):
        _gat_a(q).wait()
        _gat_b(q).wait()

    def _scat_a(q):
        return pltpu.make_async_copy(
            rows[q].at[pl.ds(0, HALF)],
            agg_sh.at[did[q].at[pl.ds(0, HALF)]], ss[q])

    def _scat_b(q):
        return pltpu.make_async_copy(
            rows[q].at[pl.ds(HALF, HALF)],
            agg_sh.at[did[q].at[pl.ds(HALF, HALF)]], st[q])

    def _step(j, q, first=False):
        _gat_wait(q)
        _scat_a(q).start(add=True)
        _scat_b(q).start(add=True)
        if not first:
            _scat_a((q + 2) % 3).wait()
            _scat_b((q + 2) % 3).wait()
        pl.when(j + 2 < CPT)(lambda: _idx_start(j + 2, (q + 2) % 3))

        def _fire_gather():
            _idx_wait(j + 1, (q + 1) % 3)
            _gat_start((q + 1) % 3)

        pl.when(j + 1 < CPT)(_fire_gather)

    _idx_start(0, 0)
    _idx_start(1, 1)
    _idx_wait(0, 0)
    _gat_start(0)

    _step(0, 0, first=True)
    _step(1, 1)

    def _body(i, _):
        for qq in range(3):
            _step(3 * i + 2 + qq, (2 + qq) % 3)
        return 0

    lax.fori_loop(0, (CPT - 2) // 3, _body, 0)
    _scat_a((CPT - 1) % 3).wait()
    _scat_b((CPT - 1) % 3).wait()

    plsc.subcore_barrier()
    pltpu.sync_copy(agg_sh.at[pl.ds(s * RPT, RPT)],
                    out_hbm.at[c, pl.ds(s * RPT, RPT)])


_BLK = 1000


def _norm_h_body(deg_ref, feat_ref, h_ref, norm_ref):
    d = deg_ref[:, 0:1] + deg_ref[:, 1:2]
    nrm = jnp.where(d == 0.0, 1.0, lax.rsqrt(jnp.maximum(d, 1.0)))
    norm_ref[...] = nrm
    h_ref[...] = feat_ref[...] * nrm


def _norm_h_call(deg_nt, feat):
    grid = (N_NODES // _BLK,)
    return pl.pallas_call(
        _norm_h_body,
        grid=grid,
        in_specs=[
            pl.BlockSpec((_BLK, 2), lambda i: (i, 0)),
            pl.BlockSpec((_BLK, D), lambda i: (i, 0)),
        ],
        out_specs=[
            pl.BlockSpec((_BLK, D), lambda i: (i, 0)),
            pl.BlockSpec((_BLK, 1), lambda i: (i, 0)),
        ],
        out_shape=[
            jax.ShapeDtypeStruct((N_NODES, D), jnp.float32),
            jax.ShapeDtypeStruct((N_NODES, 1), jnp.float32),
        ],
    )(deg_nt, feat)


def _final_body(agg_ref, feat_ref, norm_ref, w1_ref, w2_ref, bias_ref, out_ref):
    agg = agg_ref[0] + agg_ref[1]
    rst = agg * norm_ref[...]
    out_ref[...] = (
        jnp.dot(rst, w1_ref[...], preferred_element_type=jnp.float32)
        + jnp.dot(feat_ref[...], w2_ref[...], preferred_element_type=jnp.float32)
        + bias_ref[...]
    )


def _final_call(agg_parts, feat, norm, w1, w2, bias2):
    grid = (N_NODES // _BLK,)
    return pl.pallas_call(
        _final_body,
        grid=grid,
        in_specs=[
            pl.BlockSpec((NC, _BLK, D), lambda i: (0, i, 0)),
            pl.BlockSpec((_BLK, D), lambda i: (i, 0)),
            pl.BlockSpec((_BLK, 1), lambda i: (i, 0)),
            pl.BlockSpec((D, D), lambda i: (0, 0)),
            pl.BlockSpec((D, D), lambda i: (0, 0)),
            pl.BlockSpec((1, D), lambda i: (0, 0)),
        ],
        out_specs=pl.BlockSpec((_BLK, D), lambda i: (i, 0)),
        out_shape=jax.ShapeDtypeStruct((N_NODES, D), jnp.float32),
    )(agg_parts, feat, norm, w1, w2, bias2)


def kernel(feat, edge_index, weight, bias):
    src = edge_index[0]
    dst = edge_index[1]
    e = src.shape[0]
    pad_e = E_PAD - e
    pad_ar = lax.iota(jnp.int32, pad_e)
    src_p = jnp.concatenate([src, pad_ar % N_NODES])
    dst_p = jnp.concatenate([dst, N_NODES + pad_ar % (N_PAD - N_NODES)])

    deg_parts = _deg_call(dst_p.reshape(NW, CPT, CHUNK)).reshape(NC, N_PAD)
    deg_nt = deg_parts.T
    h, norm = _norm_h_call(deg_nt, feat)
    agg_parts = _agg_call(src_p, dst_p, h)
    return _final_call(agg_parts, feat, norm,
                       weight[:D], weight[D:], bias.reshape(1, D))


# speedup vs baseline: 1.0013x; 1.0013x over previous
import functools

import jax
import jax.numpy as jnp
from jax import lax
from jax.experimental import pallas as pl
from jax.experimental.pallas import tpu as pltpu
from jax.experimental.pallas import tpu_sc as plsc

N_NODES = 10000
D = 128
NC = 2
NS = 16
NW = NC * NS
CHUNK = 112
N_PAD = 10240
RPT = N_PAD // NS
CPT = 92
E_PAD = NW * CPT * CHUNK

_mesh = plsc.VectorSubcoreMesh(core_axis_name="c", subcore_axis_name="s")


@functools.partial(
    pl.kernel,
    mesh=_mesh,
    out_type=jax.ShapeDtypeStruct((NC * N_PAD,), jnp.float32),
    scratch_types=[
        pltpu.VMEM((CPT, CHUNK), jnp.int32),
        pltpu.VMEM((CHUNK,), jnp.float32),
        pltpu.VMEM((640,), jnp.float32),
        pltpu.VMEM_SHARED((N_PAD,), jnp.float32),
        pltpu.SemaphoreType.DMA,
        pltpu.SemaphoreType.DMA,
    ],
)
def _deg_call(dst_hbm, out_hbm, didx_all, ones_v, zero_v, deg_sh, ss0, ss1):
    c = lax.axis_index("c")
    s = lax.axis_index("s")
    wid = s * NC + c
    ss = (ss0, ss1)

    def _fill_zero(i, _):
        zero_v[pl.ds(i * 16, 16)] = jnp.zeros((16,), jnp.float32)
        return 0

    lax.fori_loop(0, 640 // 16, _fill_zero, 0)

    def _fill_one(i, _):
        ones_v[pl.ds(i * 16, 16)] = jnp.ones((16,), jnp.float32)
        return 0

    lax.fori_loop(0, CHUNK // 16, _fill_one, 0)

    pltpu.sync_copy(zero_v.at[pl.ds(0, RPT)], deg_sh.at[pl.ds(s * RPT, RPT)])
    pltpu.sync_copy(dst_hbm.at[wid], didx_all)
    plsc.subcore_barrier()

    def _scat(j, q):
        return pltpu.make_async_copy(
            ones_v, deg_sh.at[didx_all.at[j]], ss[q])

    _scat(0, 0).start(add=True)
    _scat(1, 1).start(add=True)

    def _body(i, _):
        j0 = 2 * i
        _scat(j0 - 2, 0).wait()
        _scat(j0, 0).start(add=True)
        _scat(j0 - 1, 1).wait()
        _scat(j0 + 1, 1).start(add=True)
        return 0

    lax.fori_loop(1, CPT // 2, _body, 0)
    _scat(CPT - 2, 0).wait()
    _scat(CPT - 1, 1).wait()

    plsc.subcore_barrier()
    pltpu.sync_copy(deg_sh.at[pl.ds(s * RPT, RPT)],
                    out_hbm.at[pl.ds(c * N_PAD + s * RPT, RPT)])


@functools.partial(
    pl.kernel,
    mesh=_mesh,
    out_type=jax.ShapeDtypeStruct((NC, N_PAD, D), jnp.float32),
    scratch_types=[
        pltpu.VMEM((CHUNK,), jnp.int32),
        pltpu.VMEM((CHUNK,), jnp.int32),
        pltpu.VMEM((CHUNK,), jnp.int32),
        pltpu.VMEM((CHUNK,), jnp.int32),
        pltpu.VMEM((CHUNK,), jnp.int32),
        pltpu.VMEM((CHUNK,), jnp.int32),
        pltpu.VMEM((CHUNK, D), jnp.float32),
        pltpu.VMEM((CHUNK, D), jnp.float32),
        pltpu.VMEM((CHUNK, D), jnp.float32),
        pltpu.VMEM_SHARED((N_PAD, D), jnp.float32),
        pltpu.SemaphoreType.DMA,
        pltpu.SemaphoreType.DMA,
        pltpu.SemaphoreType.DMA,
        pltpu.SemaphoreType.DMA,
        pltpu.SemaphoreType.DMA,
        pltpu.SemaphoreType.DMA,
        pltpu.SemaphoreType.DMA,
        pltpu.SemaphoreType.DMA,
        pltpu.SemaphoreType.DMA,
        pltpu.SemaphoreType.DMA,
        pltpu.SemaphoreType.DMA,
        pltpu.SemaphoreType.DMA,
        pltpu.SemaphoreType.DMA,
        pltpu.SemaphoreType.DMA,
        pltpu.SemaphoreType.DMA,
    ],
)
def _agg_call(src_hbm, dst_hbm, h_hbm, out_hbm, sid0, sid1, sid2, did0, did1, did2,
              rows0, rows1, rows2, agg_sh,
              si0, si1, si2, sg0, sg1, sg2, sh0, sh1, sh2,
              ss0, ss1, ss2, st0, st1, st2):
    c = lax.axis_index("c")
    s = lax.axis_index("s")
    wid = s * NC + c
    sid = (sid0, sid1, sid2)
    did = (did0, did1, did2)
    rows = (rows0, rows1, rows2)
    si = (si0, si1, si2)
    sg = (sg0, sg1, sg2)
    sh = (sh0, sh1, sh2)
    ss = (ss0, ss1, ss2)
    st = (st0, st1, st2)

    def _zrow(i, _):
        def _zcol(k, _):
            rows0[i, pl.ds(k * 16, 16)] = jnp.zeros((16,), jnp.float32)
            return 0
        return lax.fori_loop(0, D // 16, _zcol, 0)

    lax.fori_loop(0, CHUNK, _zrow, 0)
    for t in range(RPT // CHUNK):
        pltpu.sync_copy(rows0, agg_sh.at[pl.ds(s * RPT + t * CHUNK, CHUNK)])
    if RPT % CHUNK:
        _tail = RPT % CHUNK
        pltpu.sync_copy(
            rows0.at[pl.ds(0, _tail)],
            agg_sh.at[pl.ds(s * RPT + (RPT // CHUNK) * CHUNK, _tail)])
    plsc.subcore_barrier()

    def _idx_start(j, q):
        base = (wid * CPT + j) * CHUNK
        pltpu.async_copy(src_hbm.at[pl.ds(base, CHUNK)], sid[q], si[q])
        pltpu.async_copy(dst_hbm.at[pl.ds(base, CHUNK)], did[q], si[q])

    def _idx_wait(j, q):
        base = (wid * CPT + j) * CHUNK
        pltpu.make_async_copy(src_hbm.at[pl.ds(base, CHUNK)], sid[q], si[q]).wait()
        pltpu.make_async_copy(dst_hbm.at[pl.ds(base, CHUNK)], did[q], si[q]).wait()

    HALF = CHUNK // 2

    def _gat_a(q):
        return pltpu.make_async_copy(
            h_hbm.at[sid[q].at[pl.ds(0, HALF)]],
            rows[q].at[pl.ds(0, HALF)], sg[q])

    def _gat_b(q):
        return pltpu.make_async_copy(
            h_hbm.at[sid[q].at[pl.ds(HALF, HALF)]],
            rows[q].at[pl.ds(HALF, HALF)], sh[q])

    def _gat_start(q):
        _gat_a(q).start()
        _gat_b(q).start()

    def _gat_wait(q):
        _gat_a(q).wait()
        _gat_b(q).wait()

    def _scat_a(q):
        return pltpu.make_async_copy(
            rows[q].at[pl.ds(0, HALF)],
            agg_sh.at[did[q].at[pl.ds(0, HALF)]], ss[q])

    def _scat_b(q):
        return pltpu.make_async_copy(
            rows[q].at[pl.ds(HALF, HALF)],
            agg_sh.at[did[q].at[pl.ds(HALF, HALF)]], st[q])

    def _step(j, q, first=False):
        _gat_wait(q)
        _scat_a(q).start(add=True)
        _scat_b(q).start(add=True)
        if not first:
            _scat_a((q + 2) % 3).wait()
            _scat_b((q + 2) % 3).wait()
        pl.when(j + 2 < CPT)(lambda: _idx_start(j + 2, (q + 2) % 3))

        def _fire_gather():
            _idx_wait(j + 1, (q + 1) % 3)
            _gat_start((q + 1) % 3)

        pl.when(j + 1 < CPT)(_fire_gather)

    _idx_start(0, 0)
    _idx_start(1, 1)
    _idx_wait(0, 0)
    _gat_start(0)

    _step(0, 0, first=True)
    _step(1, 1)

    def _body(i, _):
        for qq in range(3):
            _step(3 * i + 2 + qq, (2 + qq) % 3)
        return 0

    lax.fori_loop(0, (CPT - 2) // 3, _body, 0)
    _scat_a((CPT - 1) % 3).wait()
    _scat_b((CPT - 1) % 3).wait()

    plsc.subcore_barrier()
    pltpu.sync_copy(agg_sh.at[pl.ds(s * RPT, RPT)],
                    out_hbm.at[c, pl.ds(s * RPT, RPT)])


_BLK = 1000


def _norm_h_body(deg_ref, feat_ref, h_ref, norm_ref):
    d = deg_ref[:, 0:1] + deg_ref[:, 1:2]
    nrm = jnp.where(d == 0.0, 1.0, lax.rsqrt(jnp.maximum(d, 1.0)))
    norm_ref[...] = nrm
    h_ref[...] = feat_ref[...] * nrm


def _norm_h_call(deg_nt, feat):
    grid = (N_NODES // _BLK,)
    return pl.pallas_call(
        _norm_h_body,
        grid=grid,
        in_specs=[
            pl.BlockSpec((_BLK, 2), lambda i: (i, 0)),
            pl.BlockSpec((_BLK, D), lambda i: (i, 0)),
        ],
        out_specs=[
            pl.BlockSpec((_BLK, D), lambda i: (i, 0)),
            pl.BlockSpec((_BLK, 1), lambda i: (i, 0)),
        ],
        out_shape=[
            jax.ShapeDtypeStruct((N_NODES, D), jnp.float32),
            jax.ShapeDtypeStruct((N_NODES, 1), jnp.float32),
        ],
    )(deg_nt, feat)


def _fb_body(feat_ref, w2_ref, bias_ref, fb_ref):
    fb_ref[...] = (
        jnp.dot(feat_ref[...], w2_ref[...], preferred_element_type=jnp.float32)
        + bias_ref[...]
    )


def _fb_call(feat, w2, bias2):
    grid = (N_NODES // _BLK,)
    return pl.pallas_call(
        _fb_body,
        grid=grid,
        in_specs=[
            pl.BlockSpec((_BLK, D), lambda i: (i, 0)),
            pl.BlockSpec((D, D), lambda i: (0, 0)),
            pl.BlockSpec((1, D), lambda i: (0, 0)),
        ],
        out_specs=pl.BlockSpec((_BLK, D), lambda i: (i, 0)),
        out_shape=jax.ShapeDtypeStruct((N_NODES, D), jnp.float32),
    )(feat, w2, bias2)


def _final_body(agg_ref, fb_ref, norm_ref, w1_ref, out_ref):
    agg = agg_ref[0] + agg_ref[1]
    rst = agg * norm_ref[...]
    out_ref[...] = (
        jnp.dot(rst, w1_ref[...], preferred_element_type=jnp.float32)
        + fb_ref[...]
    )


def _final_call(agg_parts, fb, norm, w1):
    grid = (N_NODES // _BLK,)
    return pl.pallas_call(
        _final_body,
        grid=grid,
        in_specs=[
            pl.BlockSpec((NC, _BLK, D), lambda i: (0, i, 0)),
            pl.BlockSpec((_BLK, D), lambda i: (i, 0)),
            pl.BlockSpec((_BLK, 1), lambda i: (i, 0)),
            pl.BlockSpec((D, D), lambda i: (0, 0)),
        ],
        out_specs=pl.BlockSpec((_BLK, D), lambda i: (i, 0)),
        out_shape=jax.ShapeDtypeStruct((N_NODES, D), jnp.float32),
    )(agg_parts, fb, norm, w1)


def kernel(feat, edge_index, weight, bias):
    src = edge_index[0]
    dst = edge_index[1]
    e = src.shape[0]
    pad_e = E_PAD - e
    pad_ar = lax.iota(jnp.int32, pad_e)
    src_p = jnp.concatenate([src, pad_ar % N_NODES])
    dst_p = jnp.concatenate([dst, N_NODES + pad_ar % (N_PAD - N_NODES)])

    deg_parts = _deg_call(dst_p.reshape(NW, CPT, CHUNK)).reshape(NC, N_PAD)
    deg_nt = deg_parts.T
    h, norm = _norm_h_call(deg_nt, feat)
    fb = _fb_call(feat, weight[D:], bias.reshape(1, D))
    agg_parts = _agg_call(src_p, dst_p, h)
    return _final_call(agg_parts, fb, norm, weight[:D])


# speedup vs baseline: 1.0744x; 1.0729x over previous
import functools

import jax
import jax.numpy as jnp
from jax import lax
from jax.experimental import pallas as pl
from jax.experimental.pallas import tpu as pltpu
from jax.experimental.pallas import tpu_sc as plsc

N_NODES = 10000
D = 128
NC = 2
NS = 16
NW = NC * NS
CHUNK = 125
CPT = 80
EPT = CPT * CHUNK
N_PAD = 10240
RPT = N_PAD // NS
HA = 64
HB = CHUNK - HA

_mesh = plsc.VectorSubcoreMesh(core_axis_name="c", subcore_axis_name="s")


@functools.partial(
    pl.kernel,
    mesh=_mesh,
    out_type=jax.ShapeDtypeStruct((NC, N_PAD), jnp.float32),
    scratch_types=[
        pltpu.VMEM((CPT, CHUNK), jnp.int32),
        pltpu.VMEM((128,), jnp.float32),
        pltpu.VMEM((RPT,), jnp.float32),
        pltpu.VMEM_SHARED((N_PAD,), jnp.float32),
        pltpu.SemaphoreType.DMA,
        pltpu.SemaphoreType.DMA,
    ],
)
def _deg_call(dst_hbm, out_hbm, didx_all, ones_v, zero_v, deg_sh, ss0, ss1):
    c = lax.axis_index("c")
    s = lax.axis_index("s")
    wid = s * NC + c
    ss = (ss0, ss1)

    def _fill_zero(i, _):
        zero_v[pl.ds(i * 16, 16)] = jnp.zeros((16,), jnp.float32)
        return 0

    lax.fori_loop(0, RPT // 16, _fill_zero, 0)

    def _fill_one(i, _):
        ones_v[pl.ds(i * 16, 16)] = jnp.ones((16,), jnp.float32)
        return 0

    lax.fori_loop(0, 128 // 16, _fill_one, 0)

    pltpu.sync_copy(zero_v, deg_sh.at[pl.ds(s * RPT, RPT)])
    pltpu.sync_copy(dst_hbm.at[wid], didx_all)
    plsc.subcore_barrier()

    def _scat(j, q):
        return pltpu.make_async_copy(
            ones_v.at[pl.ds(0, CHUNK)],
            deg_sh.at[didx_all.at[j]], ss[q])

    _scat(0, 0).start(add=True)
    _scat(1, 1).start(add=True)

    def _body(i, _):
        j0 = 2 * i
        _scat(j0 - 2, 0).wait()
        _scat(j0, 0).start(add=True)
        _scat(j0 - 1, 1).wait()
        _scat(j0 + 1, 1).start(add=True)
        return 0

    lax.fori_loop(1, CPT // 2, _body, 0)
    _scat(CPT - 2, 0).wait()
    _scat(CPT - 1, 1).wait()

    plsc.subcore_barrier()
    pltpu.sync_copy(deg_sh.at[pl.ds(s * RPT, RPT)],
                    out_hbm.at[c, pl.ds(s * RPT, RPT)])


@functools.partial(
    pl.kernel,
    mesh=_mesh,
    out_type=jax.ShapeDtypeStruct((NC, N_PAD, D), jnp.float32),
    scratch_types=[
        pltpu.VMEM((CHUNK,), jnp.int32),
        pltpu.VMEM((CHUNK,), jnp.int32),
        pltpu.VMEM((CHUNK,), jnp.int32),
        pltpu.VMEM((CHUNK,), jnp.int32),
        pltpu.VMEM((CHUNK,), jnp.int32),
        pltpu.VMEM((CHUNK,), jnp.int32),
        pltpu.VMEM((128, D), jnp.float32),
        pltpu.VMEM((128, D), jnp.float32),
        pltpu.VMEM_SHARED((N_PAD, D), jnp.float32),
        pltpu.SemaphoreType.DMA,
        pltpu.SemaphoreType.DMA,
        pltpu.SemaphoreType.DMA,
        pltpu.SemaphoreType.DMA,
        pltpu.SemaphoreType.DMA,
        pltpu.SemaphoreType.DMA,
        pltpu.SemaphoreType.DMA,
        pltpu.SemaphoreType.DMA,
        pltpu.SemaphoreType.DMA,
        pltpu.SemaphoreType.DMA,
        pltpu.SemaphoreType.DMA,
    ],
)
def _agg_call(src_hbm, dst_hbm, h_hbm, out_hbm, sid0, sid1, sid2,
              did0, did1, did2, rows0, rows1, agg_sh,
              si0, si1, si2, sg0, sg1, sh0, sh1, ss0, ss1, st0, st1):
    c = lax.axis_index("c")
    s = lax.axis_index("s")
    wid = s * NC + c
    sid = (sid0, sid1, sid2)
    did = (did0, did1, did2)
    rows = (rows0, rows1)
    si = (si0, si1, si2)
    sg = (sg0, sg1)
    sh = (sh0, sh1)
    ss = (ss0, ss1)
    st = (st0, st1)

    def _zrow(i, _):
        def _zcol(k, _):
            rows0[i, pl.ds(k * 16, 16)] = jnp.zeros((16,), jnp.float32)
            return 0
        return lax.fori_loop(0, D // 16, _zcol, 0)

    lax.fori_loop(0, 128, _zrow, 0)
    for t in range(RPT // 128):
        pltpu.sync_copy(rows0, agg_sh.at[pl.ds(s * RPT + t * 128, 128)])
    plsc.subcore_barrier()

    def _idx_start(j, r):
        pltpu.async_copy(src_hbm.at[wid, j], sid[r], si[r])
        pltpu.async_copy(dst_hbm.at[wid, j], did[r], si[r])

    def _idx_wait(j, r):
        pltpu.make_async_copy(src_hbm.at[wid, j], sid[r], si[r]).wait()
        pltpu.make_async_copy(dst_hbm.at[wid, j], did[r], si[r]).wait()

    def _gat_a(q, r):
        return pltpu.make_async_copy(
            h_hbm.at[sid[r].at[pl.ds(0, HA)]],
            rows[q].at[pl.ds(0, HA)], sg[q])

    def _gat_b(q, r):
        return pltpu.make_async_copy(
            h_hbm.at[sid[r].at[pl.ds(HA, HB)]],
            rows[q].at[pl.ds(HA, HB)], sh[q])

    def _gat_start(q, r):
        _gat_a(q, r).start()
        _gat_b(q, r).start()

    def _gat_wait(q, r):
        _gat_a(q, r).wait()
        _gat_b(q, r).wait()

    def _scat_a(q, r):
        return pltpu.make_async_copy(
            rows[q].at[pl.ds(0, HA)],
            agg_sh.at[did[r].at[pl.ds(0, HA)]], ss[q])

    def _scat_b(q, r):
        return pltpu.make_async_copy(
            rows[q].at[pl.ds(HA, HB)],
            agg_sh.at[did[r].at[pl.ds(HA, HB)]], st[q])

    def _step(j, q, r, first=False):
        _gat_wait(q, r)
        _scat_a(q, r).start(add=True)
        _scat_b(q, r).start(add=True)
        if not first:
            _scat_a(1 - q, (r + 2) % 3).wait()
            _scat_b(1 - q, (r + 2) % 3).wait()
        pl.when(j + 2 < CPT)(lambda: _idx_start(j + 2, (r + 2) % 3))

        def _fire_gather():
            _idx_wait(j + 1, (r + 1) % 3)
            _gat_start(1 - q, (r + 1) % 3)

        pl.when(j + 1 < CPT)(_fire_gather)

    _idx_start(0, 0)
    _idx_start(1, 1)
    _idx_wait(0, 0)
    _gat_start(0, 0)

    _step(0, 0, 0, first=True)
    _step(1, 1, 1)

    def _body(i, _):
        for k in range(6):
            _step(6 * i + 2 + k, k % 2, (2 + k) % 3)
        return 0

    lax.fori_loop(0, (CPT - 2) // 6, _body, 0)
    _scat_a((CPT - 1) % 2, (CPT - 1) % 3).wait()
    _scat_b((CPT - 1) % 2, (CPT - 1) % 3).wait()

    plsc.subcore_barrier()
    pltpu.sync_copy(agg_sh.at[pl.ds(s * RPT, RPT)],
                    out_hbm.at[c, pl.ds(s * RPT, RPT)])


_BLK = 1000


def _norm_h_body(deg_ref, feat_ref, w2_ref, bias_ref, h_ref, norm_ref, fb_ref):
    d = (deg_ref[0] + deg_ref[1])[:, None]
    nrm = jnp.where(d == 0.0, 1.0, lax.rsqrt(jnp.maximum(d, 1.0)))
    norm_ref[...] = nrm
    feat = feat_ref[...]
    h_ref[...] = feat * nrm
    fb_ref[...] = (
        jnp.dot(feat, w2_ref[...], preferred_element_type=jnp.float32)
        + bias_ref[...]
    )


def _norm_h_call(deg_parts, feat, w2, bias2):
    blk = 1280
    grid = (N_PAD // blk,)
    return pl.pallas_call(
        _norm_h_body,
        grid=grid,
        in_specs=[
            pl.BlockSpec((NC, blk), lambda i: (0, i)),
            pl.BlockSpec((blk, D), lambda i: (i, 0)),
            pl.BlockSpec((D, D), lambda i: (0, 0)),
            pl.BlockSpec((1, D), lambda i: (0, 0)),
        ],
        out_specs=[
            pl.BlockSpec((blk, D), lambda i: (i, 0)),
            pl.BlockSpec((blk, 1), lambda i: (i, 0)),
            pl.BlockSpec((blk, D), lambda i: (i, 0)),
        ],
        out_shape=[
            jax.ShapeDtypeStruct((N_NODES, D), jnp.float32),
            jax.ShapeDtypeStruct((N_NODES, 1), jnp.float32),
            jax.ShapeDtypeStruct((N_NODES, D), jnp.float32),
        ],
    )(deg_parts, feat, w2, bias2)


def _final_body(agg_ref, fb_ref, norm_ref, w1_ref, out_ref):
    agg = agg_ref[0] + agg_ref[1]
    rst = agg * norm_ref[...]
    out_ref[...] = (
        jnp.dot(rst, w1_ref[...], preferred_element_type=jnp.float32)
        + fb_ref[...]
    )


def _final_call(agg_parts, fb, norm, w1):
    grid = (N_NODES // _BLK,)
    return pl.pallas_call(
        _final_body,
        grid=grid,
        in_specs=[
            pl.BlockSpec((NC, _BLK, D), lambda i: (0, i, 0)),
            pl.BlockSpec((_BLK, D), lambda i: (i, 0)),
            pl.BlockSpec((_BLK, 1), lambda i: (i, 0)),
            pl.BlockSpec((D, D), lambda i: (0, 0)),
        ],
        out_specs=pl.BlockSpec((_BLK, D), lambda i: (i, 0)),
        out_shape=jax.ShapeDtypeStruct((N_NODES, D), jnp.float32),
    )(agg_parts, fb, norm, w1)


def kernel(feat, edge_index, weight, bias):
    src3 = edge_index[0].reshape(NW, CPT, CHUNK)
    dst3 = edge_index[1].reshape(NW, CPT, CHUNK)
    deg_parts = _deg_call(dst3)
    h, norm, fb = _norm_h_call(deg_parts, feat, weight[D:], bias.reshape(1, D))
    agg_parts = _agg_call(src3, dst3, h)
    return _final_call(agg_parts, fb, norm, weight[:D])
